# Initial kernel scaffold; baseline (speedup 1.0000x reference)
import jax
import jax.numpy as jnp
from jax.experimental import pallas as pl


def kernel(x, iInd, jInd, W):
    raise NotImplementedError("write your pallas kernel here")


# speedup vs baseline: 3.3551x; 3.3551x over previous
import jax
import jax.numpy as jnp
from jax import lax
from jax.experimental import pallas as pl
from jax.experimental.pallas import tpu as pltpu
from jax.experimental.pallas import tpu_sc as plsc

N_NODES = 10000
C = 128
N_EDGES = 320000

NC = 2
NS = 16
NW = NC * NS
K = 128
CHUNKS = -(-N_EDGES // (NW * K))
PER_W = CHUNKS * K
E_PAD = PER_W * NW
SLAB = 624
TAIL0 = N_NODES - NS * SLAB


def _sc_body(xT, iInd, jInd, W, out, acc, iidx, jidx, wbuf, rows, sem):
    cid = lax.axis_index("c")
    sid = lax.axis_index("s")
    wid = cid * NS + sid

    def zero_row(i, _):
        for j in range(C // 16):
            rows[i, pl.ds(16 * j, 16)] = jnp.zeros((16,), jnp.float32)
        return 0
    lax.fori_loop(0, K, zero_row, 0)
    r0 = sid * SLAB
    off = 0
    while off < SLAB:
        n = min(K, SLAB - off)
        pltpu.sync_copy(rows.at[pl.ds(0, n)], acc.at[pl.ds(r0 + off, n)])
        off += n

    @pl.when(sid == 0)
    def _zero_tail():
        pltpu.sync_copy(rows.at[pl.ds(0, TAIL0)],
                        acc.at[pl.ds(NS * SLAB, TAIL0)])
    plsc.subcore_barrier()

    def chunk(ch, _):
        base = wid * PER_W + ch * K
        pltpu.sync_copy(jInd.at[pl.ds(base, K)], jidx)
        pltpu.sync_copy(iInd.at[pl.ds(base, K)], iidx)
        pltpu.sync_copy(W.at[pl.ds(base, K)], wbuf)
        pltpu.async_copy(xT.at[jidx], rows, sem).wait()

        def scale(g, _):
            wv = wbuf[pl.ds(16 * g, 16)]
            w2v = wv * wv
            for l in range(16):
                e = 16 * g + l
                w2 = w2v[l]
                for j in range(C // 16):
                    rows[e, pl.ds(16 * j, 16)] = rows[e, pl.ds(16 * j, 16)] * w2
            return 0
        lax.fori_loop(0, K // 16, scale, 0)

        pltpu.sync_copy(rows, acc.at[iidx], add=True)
        return 0
    lax.fori_loop(0, CHUNKS, chunk, 0)

    plsc.subcore_barrier()
    pltpu.sync_copy(acc.at[pl.ds(r0, SLAB)], out.at[cid, pl.ds(r0, SLAB)])

    @pl.when(sid == 0)
    def _write_tail():
        pltpu.sync_copy(acc.at[pl.ds(NS * SLAB, TAIL0)],
                        out.at[cid, pl.ds(NS * SLAB, TAIL0)])


def _combine_body(p_ref, o_ref):
    s = p_ref[0] + p_ref[1]
    o_ref[0] = s.T


_combine = pl.pallas_call(
    _combine_body,
    out_shape=jax.ShapeDtypeStruct((1, C, N_NODES), jnp.float32),
)


def kernel(x, iInd, jInd, W):
    xT = jnp.swapaxes(x[0], 0, 1)
    pad = E_PAD - iInd.shape[0]
    iP = jnp.concatenate([iInd, jnp.zeros((pad,), jnp.int32)])
    jP = jnp.concatenate([jInd, jnp.zeros((pad,), jnp.int32)])
    wP = jnp.concatenate([W, jnp.zeros((pad,), jnp.float32)])

    sc = pl.kernel(
        _sc_body,
        out_type=jax.ShapeDtypeStruct((NC, N_NODES, C), jnp.float32),
        mesh=plsc.VectorSubcoreMesh(core_axis_name="c", subcore_axis_name="s"),
        scratch_types=[
            pltpu.VMEM_SHARED((N_NODES, C), jnp.float32),
            pltpu.VMEM((K,), jnp.int32),
            pltpu.VMEM((K,), jnp.int32),
            pltpu.VMEM((K,), jnp.float32),
            pltpu.VMEM((K, C), jnp.float32),
            pltpu.SemaphoreType.DMA,
        ],
    )
    partial = sc(xT, iP, jP, wP)
    return _combine(partial)
